# baseline (speedup 1.0000x reference)
import functools

import jax
import jax.numpy as jnp
from jax import lax
from jax.experimental import pallas as pl
from jax.experimental.pallas import tpu as pltpu
from jax.experimental.pallas import tpu_sc as plsc

_NUM_ROWS = 2048
_DIM = 1024
_BATCH = 4
_HEAD = 8
_LANES = 16
_NVEC = _DIM // _LANES
_BLOCK_ROWS = 256


def _sc_head_body(x_hbm, emb_hbm, out_hbm, bx, be, bp):
    wid = lax.axis_index("s") * 2 + lax.axis_index("c")
    b = wid // _HEAD
    r = wid % _HEAD
    pr = (3 * r) % _HEAD
    pltpu.sync_copy(x_hbm.at[b, r], bx)
    pltpu.sync_copy(emb_hbm.at[r], be)
    pltpu.sync_copy(emb_hbm.at[pr], bp)

    @plsc.parallel_loop(0, _NVEC, unroll=4)
    def _(k):
        col = k * _LANES
        bx[pl.ds(col, _LANES)] = (
            bx[pl.ds(col, _LANES)]
            + be[pl.ds(col, _LANES)]
            + bp[pl.ds(col, _LANES)]
        )

    pltpu.sync_copy(bx, out_hbm.at[b, r])


_sc_head_kernel = functools.partial(
    pl.kernel,
    out_type=jax.ShapeDtypeStruct((_BATCH, _HEAD, _DIM), jnp.float32),
    mesh=plsc.VectorSubcoreMesh(core_axis_name="c", subcore_axis_name="s"),
    scratch_types=[
        pltpu.VMEM((_DIM,), jnp.float32),
        pltpu.VMEM((_DIM,), jnp.float32),
        pltpu.VMEM((_DIM,), jnp.float32),
    ],
)(_sc_head_body)


def _tc_body(x_ref, e_ref, o_ref):
    o_ref[...] = x_ref[...] + 2.0 * e_ref[0][None]


def _tc_dense(x, embedding):
    grid = (_NUM_ROWS // _BLOCK_ROWS,)
    return pl.pallas_call(
        _tc_body,
        grid=grid,
        in_specs=[
            pl.BlockSpec((_BATCH, _BLOCK_ROWS, _DIM), lambda r: (0, r, 0)),
            pl.BlockSpec((1, _BLOCK_ROWS, _DIM), lambda r: (0, r, 0)),
        ],
        out_specs=pl.BlockSpec((_BATCH, _BLOCK_ROWS, _DIM), lambda r: (0, r, 0)),
        out_shape=jax.ShapeDtypeStruct(x.shape, x.dtype),
    )(x, embedding)


def kernel(x, embedding):
    emb2 = embedding.reshape(_NUM_ROWS, _DIM)
    head = _sc_head_kernel(x, emb2)
    dense = _tc_dense(x, embedding)
    return lax.dynamic_update_slice(dense, head, (0, 0, 0))


# speedup vs baseline: 1.7153x; 1.7153x over previous
import jax
import jax.numpy as jnp
from jax.experimental import pallas as pl
from jax.experimental.pallas import tpu as pltpu

_NUM_ROWS = 2048
_DIM = 1024
_BATCH = 4
_BLOCK_ROWS = 256


def _body(x_ref, e_ref, o_ref):
    e = e_ref[0]
    r = pl.program_id(0)

    @pl.when(r == 0)
    def _():
        perm_head = jnp.concatenate(
            [e[0:1], e[3:4], e[6:7], e[1:2], e[4:5], e[7:8], e[2:3], e[5:6]],
            axis=0,
        )
        esum = jnp.concatenate([e[:8] + perm_head, 2.0 * e[8:]], axis=0)
        o_ref[...] = x_ref[...] + esum[None]

    @pl.when(r != 0)
    def _():
        o_ref[...] = x_ref[...] + 2.0 * e[None]


def kernel(x, embedding):
    grid = (_NUM_ROWS // _BLOCK_ROWS,)
    return pl.pallas_call(
        _body,
        grid=grid,
        in_specs=[
            pl.BlockSpec((_BATCH, _BLOCK_ROWS, _DIM), lambda r: (0, r, 0)),
            pl.BlockSpec((1, _BLOCK_ROWS, _DIM), lambda r: (0, r, 0)),
        ],
        out_specs=pl.BlockSpec((_BATCH, _BLOCK_ROWS, _DIM), lambda r: (0, r, 0)),
        out_shape=jax.ShapeDtypeStruct(x.shape, x.dtype),
        compiler_params=pltpu.CompilerParams(
            dimension_semantics=("parallel",),
        ),
    )(x, embedding)
